# baseline (speedup 1.0000x reference)
import functools

import jax
import jax.numpy as jnp
from jax import lax
from jax.experimental import pallas as pl
from jax.experimental.pallas import tpu as pltpu
from jax.experimental.pallas import tpu_sc as plsc

B, S, V, D = 4, 8192, 8194, 128
N = B * S


_EPS = 1e-6


def _ln_rows(h, gamma, beta):
    mean = jnp.mean(h, axis=-1, keepdims=True)
    c = h - mean
    var = jnp.mean(c * c, axis=-1, keepdims=True)
    return gamma * c / jnp.sqrt(var + _EPS) + beta


def _ln_table_body(x_ref, g_ref, b_ref, o_ref):
    o_ref[...] = _ln_rows(x_ref[...], g_ref[...], b_ref[...])


def _ln_table_pack_body(x_ref, g_ref, b_ref, o_ref):
    ln = _ln_rows(x_ref[...], g_ref[...], b_ref[...])
    lo = jax.lax.bitcast_convert_type(ln[:, :D // 2].astype(jnp.bfloat16),
                                      jnp.uint16).astype(jnp.uint32)
    hi = jax.lax.bitcast_convert_type(ln[:, D // 2:].astype(jnp.bfloat16),
                                      jnp.uint16).astype(jnp.uint32)
    o_ref[...] = jax.lax.bitcast_convert_type(lo | (hi << 16), jnp.int32)


def _ln_table_packed(table, gamma2, beta2, block):
    rows = table.shape[0]
    grid = pl.cdiv(rows, block)
    return pl.pallas_call(
        _ln_table_pack_body,
        grid=(grid,),
        in_specs=[
            pl.BlockSpec((block, D), lambda i: (i, 0)),
            pl.BlockSpec((1, D), lambda i: (0, 0)),
            pl.BlockSpec((1, D), lambda i: (0, 0)),
        ],
        out_specs=pl.BlockSpec((block, D // 2), lambda i: (i, 0)),
        out_shape=jax.ShapeDtypeStruct((rows, D // 2), jnp.int32),
    )(table, gamma2, beta2)


def _ln_table(table, gamma2, beta2, block):
    rows = table.shape[0]
    grid = pl.cdiv(rows, block)
    return pl.pallas_call(
        _ln_table_body,
        grid=(grid,),
        in_specs=[
            pl.BlockSpec((block, D), lambda i: (i, 0)),
            pl.BlockSpec((1, D), lambda i: (0, 0)),
            pl.BlockSpec((1, D), lambda i: (0, 0)),
        ],
        out_specs=pl.BlockSpec((block, D), lambda i: (i, 0)),
        out_shape=jax.ShapeDtypeStruct((rows, D), jnp.float32),
    )(table, gamma2, beta2)


def _ln_bcast_body(er_ref, pm_ref, g_ref, b_ref, oe_ref, op_ref):
    g, b = g_ref[...], b_ref[...]
    ln_er = _ln_rows(er_ref[...], g, b)
    ln_pm = _ln_rows(pm_ref[...], g, b)
    oe_ref[...] = jnp.broadcast_to(ln_er[None], (B,) + ln_er.shape)
    op_ref[...] = jnp.broadcast_to(ln_pm[None], (B,) + ln_pm.shape)


def _ln_bcast(er, pm, gamma2, beta2, block):
    rows = er.shape[0]
    grid = rows // block
    out = jax.ShapeDtypeStruct((B, rows, D), jnp.float32)
    return pl.pallas_call(
        _ln_bcast_body,
        grid=(grid,),
        in_specs=[
            pl.BlockSpec((block, D), lambda i: (i, 0)),
            pl.BlockSpec((block, D), lambda i: (i, 0)),
            pl.BlockSpec((1, D), lambda i: (0, 0)),
            pl.BlockSpec((1, D), lambda i: (0, 0)),
        ],
        out_specs=[
            pl.BlockSpec((B, block, D), lambda i: (0, i, 0)),
            pl.BlockSpec((B, block, D), lambda i: (0, i, 0)),
        ],
        out_shape=[out, out],
    )(er, pm, gamma2, beta2)


_NC, _NS = 2, 16
_NW = _NC * _NS
_CHUNK = 128
_IDX_ROWS = N // (_NW * _CHUNK)


_NBUF = 6


_PER_W = _IDX_ROWS * _CHUNK
_NRAW = 3
_NDST = 2


def _unpack_chunk(raw, dst):
    def blk(k, carry):
        r0 = k * 16
        for r in range(16):
            for c in range(D // 2 // 16):
                v = raw[r0 + r, pl.ds(c * 16, 16)]
                dst[r0 + r, pl.ds(c * 16, 16)] = v << 16
                dst[r0 + r, pl.ds(D // 2 + c * 16, 16)] = v & jnp.int32(-65536)
        return carry
    lax.fori_loop(0, _CHUNK // 16, blk, 0)


def _gather_body(table_hbm, idx_hbm, out_hbm, idx_v,
                 r0, r1, r2, d0, d1, g0, g1, g2, o0, o1):
    wid = lax.axis_index("s") * _NC + lax.axis_index("c")
    b = wid // (S // _PER_W)
    off = (wid % (S // _PER_W)) * _PER_W
    pltpu.sync_copy(idx_hbm.at[b, pl.ds(off, _PER_W)], idx_v)
    base = b * S + off
    raws = (r0, r1, r2)
    dsts = (d0, d1)
    gsems = (g0, g1, g2)
    osems = (o0, o1)
    gathers = [None] * _IDX_ROWS
    outs = [None] * _IDX_ROWS
    for j in range(min(_NRAW, _IDX_ROWS)):
        gathers[j] = pltpu.async_copy(
            table_hbm.at[idx_v.at[pl.ds(j * _CHUNK, _CHUNK)]], raws[j], gsems[j])
    for j in range(_IDX_ROWS):
        gathers[j].wait()
        if j >= _NDST:
            outs[j - _NDST].wait()
        _unpack_chunk(raws[j % _NRAW], dsts[j % _NDST])
        outs[j] = pltpu.async_copy(
            dsts[j % _NDST], out_hbm.at[pl.ds(base + j * _CHUNK, _CHUNK), :],
            osems[j % _NDST])
        nxt = j + _NRAW
        if nxt < _IDX_ROWS:
            gathers[nxt] = pltpu.async_copy(
                table_hbm.at[idx_v.at[pl.ds(nxt * _CHUNK, _CHUNK)]],
                raws[nxt % _NRAW], gsems[nxt % _NRAW])
    for j in range(max(0, _IDX_ROWS - _NDST), _IDX_ROWS):
        outs[j].wait()


@functools.lru_cache(maxsize=1)
def _gather():
    return functools.partial(
        pl.kernel,
        mesh=plsc.VectorSubcoreMesh(core_axis_name="c", subcore_axis_name="s"),
        compiler_params=pltpu.CompilerParams(use_tc_tiling_on_sc=False),
        out_type=jax.ShapeDtypeStruct((N, D), jnp.int32),
        scratch_types=[
            pltpu.VMEM((_PER_W,), jnp.int32),
        ] + [pltpu.VMEM((_CHUNK, D // 2), jnp.int32) for _ in range(_NRAW)]
          + [pltpu.VMEM((_CHUNK, D), jnp.int32) for _ in range(_NDST)]
          + [pltpu.SemaphoreType.DMA for _ in range(_NRAW + _NDST)],
    )(_gather_body)


def kernel(x, er_embed, pm_embed, token_table, pos_table, gamma, beta):
    del pos_table
    gamma2 = gamma.reshape(1, D)
    beta2 = beta.reshape(1, D)
    norm_table = _ln_table_packed(token_table, gamma2, beta2, block=4096)
    er_out, pm_out = _ln_bcast(er_embed, pm_embed, gamma2, beta2, block=4096)
    tok = jax.lax.bitcast_convert_type(_gather()(norm_table, x), jnp.float32)
    return tok.reshape(B, S, D), er_out, pm_out


# speedup vs baseline: 1.4585x; 1.4585x over previous
import functools

import jax
import jax.numpy as jnp
from jax import lax
from jax.experimental import pallas as pl
from jax.experimental.pallas import tpu as pltpu
from jax.experimental.pallas import tpu_sc as plsc

B, S, V, D = 4, 8192, 8194, 128
N = B * S


_EPS = 1e-6


def _ln_rows(h, gamma, beta):
    mean = jnp.mean(h, axis=-1, keepdims=True)
    c = h - mean
    var = jnp.mean(c * c, axis=-1, keepdims=True)
    return gamma * c / jnp.sqrt(var + _EPS) + beta


def _ln_table_body(x_ref, g_ref, b_ref, o_ref):
    o_ref[...] = _ln_rows(x_ref[...], g_ref[...], b_ref[...])


def _ln_table(table, gamma2, beta2, block):
    rows = table.shape[0]
    grid = pl.cdiv(rows, block)
    return pl.pallas_call(
        _ln_table_body,
        grid=(grid,),
        in_specs=[
            pl.BlockSpec((block, D), lambda i: (i, 0)),
            pl.BlockSpec((1, D), lambda i: (0, 0)),
            pl.BlockSpec((1, D), lambda i: (0, 0)),
        ],
        out_specs=pl.BlockSpec((block, D), lambda i: (i, 0)),
        out_shape=jax.ShapeDtypeStruct((rows, D), jnp.float32),
    )(table, gamma2, beta2)


def _ln_bcast_body(er_ref, pm_ref, g_ref, b_ref, oe_ref, op_ref):
    g, b = g_ref[...], b_ref[...]
    ln_er = _ln_rows(er_ref[...], g, b)
    ln_pm = _ln_rows(pm_ref[...], g, b)
    oe_ref[...] = jnp.broadcast_to(ln_er[None], (B,) + ln_er.shape)
    op_ref[...] = jnp.broadcast_to(ln_pm[None], (B,) + ln_pm.shape)


def _ln_bcast(er, pm, gamma2, beta2, block):
    rows = er.shape[0]
    grid = rows // block
    out = jax.ShapeDtypeStruct((B, rows, D), jnp.float32)
    return pl.pallas_call(
        _ln_bcast_body,
        grid=(grid,),
        in_specs=[
            pl.BlockSpec((block, D), lambda i: (i, 0)),
            pl.BlockSpec((block, D), lambda i: (i, 0)),
            pl.BlockSpec((1, D), lambda i: (0, 0)),
            pl.BlockSpec((1, D), lambda i: (0, 0)),
        ],
        out_specs=[
            pl.BlockSpec((B, block, D), lambda i: (0, i, 0)),
            pl.BlockSpec((B, block, D), lambda i: (0, i, 0)),
        ],
        out_shape=[out, out],
    )(er, pm, gamma2, beta2)


_NC, _NS = 2, 16
_NW = _NC * _NS
_CHUNK = 256
_IDX_ROWS = N // (_NW * _CHUNK)


_NBUF = 3


_PER_W = _IDX_ROWS * _CHUNK


def _gather_body(table_hbm, idx_hbm, out_hbm, idx_v,
                 b0, b1, b2, g0, g1, g2, o0, o1, o2):
    wid = lax.axis_index("s") * _NC + lax.axis_index("c")
    b = wid // (S // _PER_W)
    off = (wid % (S // _PER_W)) * _PER_W
    pltpu.sync_copy(idx_hbm.at[b, pl.ds(off, _PER_W)], idx_v)
    base = b * S + off
    bufs = (b0, b1, b2)
    gsems = (g0, g1, g2)
    osems = (o0, o1, o2)
    gathers = [None] * _IDX_ROWS
    outs = [None] * _IDX_ROWS
    for j in range(min(_NBUF, _IDX_ROWS)):
        gathers[j] = pltpu.async_copy(
            table_hbm.at[idx_v.at[pl.ds(j * _CHUNK, _CHUNK)]], bufs[j], gsems[j])
    for j in range(_IDX_ROWS):
        gathers[j].wait()
        outs[j] = pltpu.async_copy(
            bufs[j % _NBUF], out_hbm.at[pl.ds(base + j * _CHUNK, _CHUNK), :],
            osems[j % _NBUF])
        nxt = j + _NBUF
        if nxt < _IDX_ROWS:
            outs[nxt - _NBUF].wait()
            gathers[nxt] = pltpu.async_copy(
                table_hbm.at[idx_v.at[pl.ds(nxt * _CHUNK, _CHUNK)]],
                bufs[nxt % _NBUF], gsems[nxt % _NBUF])
    for j in range(max(0, _IDX_ROWS - _NBUF), _IDX_ROWS):
        outs[j].wait()


@functools.lru_cache(maxsize=1)
def _gather():
    return functools.partial(
        pl.kernel,
        mesh=plsc.VectorSubcoreMesh(core_axis_name="c", subcore_axis_name="s"),
        out_type=jax.ShapeDtypeStruct((N, D), jnp.float32),
        scratch_types=[
            pltpu.VMEM((_PER_W,), jnp.int32),
        ] + [pltpu.VMEM((_CHUNK, D), jnp.float32) for _ in range(_NBUF)]
          + [pltpu.SemaphoreType.DMA for _ in range(2 * _NBUF)],
    )(_gather_body)


def kernel(x, er_embed, pm_embed, token_table, pos_table, gamma, beta):
    del pos_table
    gamma2 = gamma.reshape(1, D)
    beta2 = beta.reshape(1, D)
    norm_table = _ln_table(token_table, gamma2, beta2, block=4096)
    er_out, pm_out = _ln_bcast(er_embed, pm_embed, gamma2, beta2, block=4096)
    tok = _gather()(norm_table, x)
    return tok.reshape(B, S, D), er_out, pm_out


# speedup vs baseline: 1.4624x; 1.0027x over previous
import functools

import jax
import jax.numpy as jnp
from jax import lax
from jax.experimental import pallas as pl
from jax.experimental.pallas import tpu as pltpu
from jax.experimental.pallas import tpu_sc as plsc

B, S, V, D = 4, 8192, 8194, 128
N = B * S


_EPS = 1e-6


def _ln_rows(h, gamma, beta):
    mean = jnp.mean(h, axis=-1, keepdims=True)
    c = h - mean
    var = jnp.mean(c * c, axis=-1, keepdims=True)
    return gamma * c / jnp.sqrt(var + _EPS) + beta


def _ln_table_body(x_ref, g_ref, b_ref, o_ref):
    o_ref[...] = _ln_rows(x_ref[...], g_ref[...], b_ref[...])


def _ln_table(table, gamma2, beta2, block):
    rows = table.shape[0]
    grid = pl.cdiv(rows, block)
    return pl.pallas_call(
        _ln_table_body,
        grid=(grid,),
        in_specs=[
            pl.BlockSpec((block, D), lambda i: (i, 0)),
            pl.BlockSpec((1, D), lambda i: (0, 0)),
            pl.BlockSpec((1, D), lambda i: (0, 0)),
        ],
        out_specs=pl.BlockSpec((block, D), lambda i: (i, 0)),
        out_shape=jax.ShapeDtypeStruct((rows, D), jnp.float32),
    )(table, gamma2, beta2)


def _ln_bcast_body(er_ref, pm_ref, g_ref, b_ref, oe_ref, op_ref):
    g, b = g_ref[...], b_ref[...]
    ln_er = _ln_rows(er_ref[...], g, b)
    ln_pm = _ln_rows(pm_ref[...], g, b)
    oe_ref[...] = jnp.broadcast_to(ln_er[None], (B,) + ln_er.shape)
    op_ref[...] = jnp.broadcast_to(ln_pm[None], (B,) + ln_pm.shape)


def _ln_bcast(er, pm, gamma2, beta2, block):
    rows = er.shape[0]
    grid = rows // block
    out = jax.ShapeDtypeStruct((B, rows, D), jnp.float32)
    return pl.pallas_call(
        _ln_bcast_body,
        grid=(grid,),
        in_specs=[
            pl.BlockSpec((block, D), lambda i: (i, 0)),
            pl.BlockSpec((block, D), lambda i: (i, 0)),
            pl.BlockSpec((1, D), lambda i: (0, 0)),
            pl.BlockSpec((1, D), lambda i: (0, 0)),
        ],
        out_specs=[
            pl.BlockSpec((B, block, D), lambda i: (0, i, 0)),
            pl.BlockSpec((B, block, D), lambda i: (0, i, 0)),
        ],
        out_shape=[out, out],
    )(er, pm, gamma2, beta2)


_NC, _NS = 2, 16
_NW = _NC * _NS
_CHUNK = 128
_IDX_ROWS = N // (_NW * _CHUNK)


_NBUF = 6


_PER_W = _IDX_ROWS * _CHUNK


def _gather_body(table_hbm, idx_hbm, out_hbm, idx_v,
                 b0, b1, b2, b3, b4, b5,
                 g0, g1, g2, g3, g4, g5, o0, o1, o2, o3, o4, o5):
    wid = lax.axis_index("s") * _NC + lax.axis_index("c")
    b = wid // (S // _PER_W)
    off = (wid % (S // _PER_W)) * _PER_W
    pltpu.sync_copy(idx_hbm.at[b, pl.ds(off, _PER_W)], idx_v)
    base = b * S + off
    bufs = (b0, b1, b2, b3, b4, b5)
    gsems = (g0, g1, g2, g3, g4, g5)
    osems = (o0, o1, o2, o3, o4, o5)
    gathers = [None] * _IDX_ROWS
    outs = [None] * _IDX_ROWS
    for j in range(min(_NBUF, _IDX_ROWS)):
        gathers[j] = pltpu.async_copy(
            table_hbm.at[idx_v.at[pl.ds(j * _CHUNK, _CHUNK)]], bufs[j], gsems[j])
    for j in range(_IDX_ROWS):
        gathers[j].wait()
        outs[j] = pltpu.async_copy(
            bufs[j % _NBUF], out_hbm.at[pl.ds(base + j * _CHUNK, _CHUNK), :],
            osems[j % _NBUF])
        nxt = j + _NBUF
        if nxt < _IDX_ROWS:
            outs[nxt - _NBUF].wait()
            gathers[nxt] = pltpu.async_copy(
                table_hbm.at[idx_v.at[pl.ds(nxt * _CHUNK, _CHUNK)]],
                bufs[nxt % _NBUF], gsems[nxt % _NBUF])
    for j in range(max(0, _IDX_ROWS - _NBUF), _IDX_ROWS):
        outs[j].wait()


@functools.lru_cache(maxsize=1)
def _gather():
    return functools.partial(
        pl.kernel,
        mesh=plsc.VectorSubcoreMesh(core_axis_name="c", subcore_axis_name="s"),
        out_type=jax.ShapeDtypeStruct((N, D), jnp.float32),
        scratch_types=[
            pltpu.VMEM((_PER_W,), jnp.int32),
        ] + [pltpu.VMEM((_CHUNK, D), jnp.float32) for _ in range(_NBUF)]
          + [pltpu.SemaphoreType.DMA for _ in range(2 * _NBUF)],
    )(_gather_body)


def kernel(x, er_embed, pm_embed, token_table, pos_table, gamma, beta):
    del pos_table
    gamma2 = gamma.reshape(1, D)
    beta2 = beta.reshape(1, D)
    norm_table = _ln_table(token_table, gamma2, beta2, block=4096)
    er_out, pm_out = _ln_bcast(er_embed, pm_embed, gamma2, beta2, block=4096)
    tok = _gather()(norm_table, x)
    return tok.reshape(B, S, D), er_out, pm_out
